# baseline (speedup 1.0000x reference)
import functools

import jax
import jax.numpy as jnp
from jax import lax
from jax.experimental import pallas as pl
from jax.experimental.pallas import tpu as pltpu, tpu_sc as plsc

_V = 8192
_D = 64
_B = 4096
_S = 200
_N = _B * _S
_NW = 32
_PER_W = _N // _NW
_CHUNK = 400
_RPC = _CHUNK // _S
_NCH = _PER_W // _CHUNK
_JROWS = 40
_NJ = _S // _JROWS
_ROWS_PER_TILE = _V // 16
_MULT = 1315423911


def _body(xs_hbm, tab_hbm, g_hbm, out_hbm,
          cur_v, idx_v, rows_v, g_v, tab_sh, sem_g, sem_x, sem_s):
    c = lax.axis_index("c")
    s = lax.axis_index("s")
    wid = s * 2 + c
    lane = lax.iota(jnp.int32, 16)
    i32 = jnp.int32

    pltpu.sync_copy(g_hbm, g_v)
    sg = []
    for c4 in range(4):
        gv = g_v[pl.ds(c4 * 16, 16)]
        sg.append(1.0 / (1.0 + jnp.exp(-gv)))

    row0 = s * i32(_ROWS_PER_TILE)
    stage = rows_v.at[i32(0), i32(0)]
    half = _ROWS_PER_TILE // 4
    for hh in range(4):
        pltpu.sync_copy(tab_hbm.at[pl.ds(row0 + i32(hh * half), half)],
                        stage.at[pl.ds(0, half)])

        def _scale_row(r, _):
            for c4 in range(4):
                stage[r, pl.ds(c4 * 16, 16)] = stage[r, pl.ds(c4 * 16, 16)] * sg[c4]
            return 0

        lax.fori_loop(i32(0), i32(half), _scale_row, 0)
        pltpu.sync_copy(stage.at[pl.ds(0, half)],
                        tab_sh.at[pl.ds(row0 + i32(hh * half), half)])
    plsc.subcore_barrier()

    def _chunk(t, _):
        b = t & i32(1)
        nb = i32(1) - b
        base = wid * i32(_PER_W) + t * i32(_CHUNK)
        boff = b * i32(_CHUNK)

        @pl.when(t == i32(0))
        def _prime():
            pltpu.async_copy(xs_hbm.at[pl.ds(base, _CHUNK)],
                             cur_v.at[pl.ds(boff, _CHUNK)], sem_x.at[b])

        pltpu.make_async_copy(xs_hbm.at[pl.ds(0, _CHUNK)],
                              cur_v.at[pl.ds(boff, _CHUNK)], sem_x.at[b]).wait()

        @pl.when(t + i32(1) < i32(_NCH))
        def _prefetch():
            pltpu.async_copy(
                xs_hbm.at[pl.ds(base + i32(_CHUNK), _CHUNK)],
                cur_v.at[pl.ds(i32(_CHUNK) - boff, _CHUNK)], sem_x.at[nb])

        def _hash(kk, _):
            pos0 = kk * i32(16)
            pos = pos0 + lane
            cur = cur_v[pl.ds(boff + pos0, 16)]
            prevraw = plsc.load_gather(cur_v, [boff + jnp.maximum(pos - 1, 0)])
            col0 = ((base + pos) % i32(_S)) == i32(0)
            prev = jnp.where(col0, 0, prevraw)
            idx_v[b, pl.ds(pos0, 16)] = (prev * i32(_MULT) + cur) & i32(_V - 1)
            return 0

        lax.fori_loop(i32(0), i32(_CHUNK // 16), _hash, 0)

        @pl.when(t >= i32(2))
        def _drain_scatter():
            pltpu.make_async_copy(rows_v.at[b], out_hbm.at[pl.ds(0, _RPC)],
                                  sem_s.at[b]).wait()

        copies = []
        for rr in range(_RPC):
            for k in range(_NJ):
                off = rr * _S + k * _JROWS
                copies.append(pltpu.async_copy(
                    tab_sh.at[idx_v.at[b, pl.ds(off, _JROWS)]],
                    rows_v.at[b, i32(rr), pl.ds(k * _JROWS, _JROWS)], sem_g))
        for cp in copies:
            cp.wait()

        grow = wid * i32(_PER_W // _S) + t * i32(_RPC)
        pltpu.async_copy(rows_v.at[b], out_hbm.at[pl.ds(grow, _RPC)],
                         sem_s.at[b])
        return 0

    lax.fori_loop(i32(0), i32(_NCH), _chunk, 0)

    for bb in range(2):
        pltpu.make_async_copy(rows_v.at[jnp.int32(bb)],
                              out_hbm.at[pl.ds(0, _RPC)],
                              sem_s.at[jnp.int32(bb)]).wait()


_call = pl.kernel(
    _body,
    out_type=jax.ShapeDtypeStruct((_B, _S, _D), jnp.float32),
    mesh=plsc.VectorSubcoreMesh(core_axis_name="c", subcore_axis_name="s"),
    scratch_types=[
        pltpu.VMEM((2 * _CHUNK,), jnp.int32),
        pltpu.VMEM((2, _CHUNK), jnp.int32),
        pltpu.VMEM((2, _RPC, _S, _D), jnp.float32),
        pltpu.VMEM((_D,), jnp.float32),
        pltpu.VMEM_SHARED((_V, _D), jnp.float32),
        pltpu.SemaphoreType.DMA,
        pltpu.SemaphoreType.DMA((2,)),
        pltpu.SemaphoreType.DMA((2,)),
    ],
    compiler_params=pltpu.CompilerParams(use_tc_tiling_on_sc=False,
                                         needs_layout_passes=False),
)


@jax.jit
def kernel(x, emb, g):
    xs = x.astype(jnp.int32).reshape(-1)
    return _call(xs, emb.astype(jnp.float32), g.astype(jnp.float32))
